# baseline (speedup 1.0000x reference)
import dataclasses
import functools

import jax
import jax.numpy as jnp
from jax import lax
from jax.experimental import pallas as pl
from jax.experimental.pallas import tpu as pltpu
from jax.experimental.pallas import tpu_sc as plsc

_N = 10000
_E = 320000
_G = 16
_H = 64

_NC = 2
_NS = 16
_NW = _NC * _NS

_NPAD = 10112
_RPT = _NPAD // _NS
_WIN = 125
_NWIN = 80
_EHT = _E // _NW
_L = 16
_NBUF = 4


def _sc_mesh():
    return plsc.VectorSubcoreMesh(core_axis_name="c", subcore_axis_name="s")


def _sc_params(tc_tiling=True):
    cp = pltpu.CompilerParams()
    if "needs_layout_passes" in pltpu.CompilerParams.__dataclass_fields__:
        cp = dataclasses.replace(cp, needs_layout_passes=False)
    if not tc_tiling:
        cp = dataclasses.replace(cp, use_tc_tiling_on_sc=False)
    return cp


def _degree_hist(ei2):
    @functools.partial(
        pl.kernel,
        out_type=jax.ShapeDtypeStruct((_NW, _NPAD), jnp.float32),
        mesh=_sc_mesh(),
        scratch_types=[
            pltpu.VMEM((_NPAD,), jnp.float32),
            pltpu.VMEM((_EHT,), jnp.int32),
        ],
        compiler_params=_sc_params(tc_tiling=False),
    )
    def body(ei_hbm, out_hbm, hist, dstv):
        c = lax.axis_index("c")
        s = lax.axis_index("s")
        wid = s * _NC + c
        pltpu.sync_copy(ei_hbm.at[1, pl.ds(wid * _EHT, _EHT)], dstv)

        zero16 = jnp.zeros((_L,), jnp.float32)

        @pl.loop(0, _NPAD, step=_L)
        def _(i):
            hist[pl.ds(i, _L)] = zero16

        one16 = jnp.ones((_L,), jnp.float32)

        @pl.loop(0, _EHT, step=_L)
        def _(i):
            idx = dstv[pl.ds(i, _L)]
            plsc.addupdate_scatter(hist, [idx], one16)

        pltpu.sync_copy(hist, out_hbm.at[wid])

    return body(ei2)


def _edge_aggregate(g, ei4, zrow):
    @functools.partial(
        pl.kernel,
        out_type=jax.ShapeDtypeStruct((_NC, _NPAD, _H), jnp.float32),
        mesh=_sc_mesh(),
        scratch_types=[
            pltpu.VMEM_SHARED((_NPAD, _H), jnp.float32),
            pltpu.VMEM((_NWIN, _WIN), jnp.int32),
            pltpu.VMEM((_NWIN, _WIN), jnp.int32),
            [pltpu.VMEM((_WIN, _H), jnp.float32) for _ in range(_NBUF)],
            [pltpu.SemaphoreType.DMA for _ in range(_NBUF)],
            [pltpu.SemaphoreType.DMA for _ in range(_NBUF)],
        ],
        compiler_params=_sc_params(tc_tiling=False),
    )
    def body(g_hbm, ei_hbm, z_hbm, out_hbm,
             acc, srcv, dstv, rows, gsem, ssem):
        c = lax.axis_index("c")
        s = lax.axis_index("s")
        wid = s * _NC + c

        pltpu.sync_copy(z_hbm, acc.at[pl.ds(s * _RPT, _RPT)])
        pltpu.sync_copy(ei_hbm.at[0, wid], srcv)
        pltpu.sync_copy(ei_hbm.at[1, wid], dstv)
        plsc.subcore_barrier()

        def gather(w, b):
            pltpu.async_copy(g_hbm.at[srcv.at[w]], rows[b], gsem[b])

        def gwait(w, b):
            pltpu.make_async_copy(g_hbm.at[srcv.at[w]], rows[b], gsem[b]).wait()

        def scat(w, b):
            pltpu.async_copy(rows[b], acc.at[dstv.at[w]], ssem[b], add=True)

        def swait(w, b):
            pltpu.make_async_copy(rows[b], acc.at[dstv.at[w]], ssem[b]).wait()

        for b in range(_NBUF):
            gather(b, b)

        @pl.loop(0, _NWIN - _NBUF, step=_NBUF)
        def _(i):
            for b in range(_NBUF):
                gwait(i + b, b)
                scat(i + b, b)
            for b in range(_NBUF):
                swait(i + b, b)
                gather(i + _NBUF + b, b)

        for b in range(_NBUF):
            gwait(_NWIN - _NBUF + b, b)
            scat(_NWIN - _NBUF + b, b)
        for b in range(_NBUF):
            swait(_NWIN - _NBUF + b, b)

        plsc.subcore_barrier()
        pltpu.sync_copy(acc.at[pl.ds(s * _RPT, _RPT)],
                        out_hbm.at[c].at[pl.ds(s * _RPT, _RPT)])

    return body(g, ei4, zrow)


def _graph_norm(xv, P, cinv, w, b, ms):
    mean = jnp.dot(P.T, xv, preferred_element_type=jnp.float32) * cinv
    out = xv - jnp.dot(P, mean, preferred_element_type=jnp.float32) * ms
    var = jnp.dot(P.T, out * out, preferred_element_type=jnp.float32) * cinv
    scale = lax.rsqrt(jnp.dot(P, var, preferred_element_type=jnp.float32) + 1e-5)
    return w * out * scale + b


def _one_hot(batch2):
    P = (batch2 == lax.broadcasted_iota(jnp.int32, (1, _G), 1))
    P = P.astype(jnp.float32)
    cnt = jnp.sum(P, axis=0).reshape(_G, 1)
    return P, 1.0 / jnp.maximum(cnt, 1.0)


def _stage_a(hist_all, x, W0):
    def body(h_ref, x_ref, w_ref, dinv_ref, g0_ref):
        deg = jnp.sum(h_ref[...], axis=0) + 1.0
        dinv = lax.rsqrt(deg)
        dinv2 = dinv[:_N].reshape(_N, 1)
        dinv_ref[...] = dinv2
        h2 = jnp.dot(x_ref[...], w_ref[...], preferred_element_type=jnp.float32)
        g0_ref[...] = h2 * dinv2

    return pl.pallas_call(
        body,
        out_shape=(jax.ShapeDtypeStruct((_N, 1), jnp.float32),
                   jax.ShapeDtypeStruct((_N, _H), jnp.float32)),
    )(hist_all, x, W0)


def _stage_b(acc, g, hprev, dinv, batch2, bias, gnw, gnb, gnms, Wn):
    residual = hprev is not None

    def body(*refs):
        if residual:
            (acc_ref, g_ref, hp_ref, dinv_ref, b_ref, bias_ref,
             gnw_ref, gnb_ref, gnms_ref, wn_ref, h_ref, gn_ref) = refs
        else:
            (acc_ref, g_ref, dinv_ref, b_ref, bias_ref,
             gnw_ref, gnb_ref, gnms_ref, wn_ref, h_ref, gn_ref) = refs
        dinv2 = dinv_ref[...]
        a = acc_ref[0, :_N, :] + acc_ref[1, :_N, :] + g_ref[...]
        pre = a * dinv2 + bias_ref[...]
        P, cinv = _one_hot(b_ref[...])
        y = _graph_norm(pre, P, cinv, gnw_ref[...], gnb_ref[...], gnms_ref[...])
        h = jnp.maximum(y, 0.0)
        if residual:
            h = h + hp_ref[...]
        h_ref[...] = h
        gn_ref[...] = jnp.dot(
            h, wn_ref[...], preferred_element_type=jnp.float32) * dinv2

    args = (acc, g) + ((hprev,) if residual else ()) + (
        dinv, batch2, bias, gnw, gnb, gnms, Wn)
    return pl.pallas_call(
        body,
        out_shape=(jax.ShapeDtypeStruct((_N, _H), jnp.float32),
                   jax.ShapeDtypeStruct((_N, _H), jnp.float32)),
    )(*args)


def _stage_final(acc, g, dinv, batch2, bias, gnw, gnb, gnms, P1, pb1, P2, pb2):
    def body(acc_ref, g_ref, dinv_ref, b_ref, bias_ref,
             gnw_ref, gnb_ref, gnms_ref, p1_ref, pb1_ref, p2_ref, pb2_ref,
             out_ref):
        dinv2 = dinv_ref[...]
        a = acc_ref[0, :_N, :] + acc_ref[1, :_N, :] + g_ref[...]
        pre = a * dinv2 + bias_ref[...]
        P, cinv = _one_hot(b_ref[...])
        y = _graph_norm(pre, P, cinv, gnw_ref[...], gnb_ref[...], gnms_ref[...])
        h = jnp.maximum(y, 0.0)
        z = jnp.maximum(
            jnp.dot(h, p1_ref[...], preferred_element_type=jnp.float32)
            + pb1_ref[...], 0.0)
        t = jnp.dot(z, p2_ref[...], preferred_element_type=jnp.float32) + pb2_ref[...]
        pos = 1.0 / (1.0 + jnp.exp(-t))
        et = jnp.exp(t)
        neg = et / (1.0 + et)
        out_ref[...] = jnp.where(t >= 0, pos, neg)

    return pl.pallas_call(
        body,
        out_shape=jax.ShapeDtypeStruct((_N, 1), jnp.float32),
    )(acc, g, dinv, batch2, bias, gnw, gnb, gnms, P1, pb1, P2, pb2)


def kernel(x, edge_index, batch, W0, b0, W1, b1, W2, b2, Wl, bl,
           gn0_w, gn0_b, gn0_ms, gn1_w, gn1_b, gn1_ms, gn2_w, gn2_b, gn2_ms,
           gn3_w, gn3_b, gn3_ms, P1, pb1, P2, pb2):
    ei4 = edge_index.reshape(2, _NW, _NWIN, _WIN)
    ei2 = ei4.reshape(2, _E)
    zrow = jnp.zeros((_RPT, _H), jnp.float32)
    batch2 = batch.reshape(_N, 1)

    hist = _degree_hist(ei2)
    dinv, g = _stage_a(hist, x, W0)

    acc = _edge_aggregate(g, ei4, zrow)
    h, g = _stage_b(acc, g, None, dinv, batch2, b0, gn0_w, gn0_b, gn0_ms, W1)

    acc = _edge_aggregate(g, ei4, zrow)
    h, g = _stage_b(acc, g, h, dinv, batch2, b1, gn1_w, gn1_b, gn1_ms, W2)

    acc = _edge_aggregate(g, ei4, zrow)
    h, g = _stage_b(acc, g, h, dinv, batch2, b2, gn2_w, gn2_b, gn2_ms, Wl)

    acc = _edge_aggregate(g, ei4, zrow)
    pred = _stage_final(acc, g, dinv, batch2, bl, gn3_w, gn3_b, gn3_ms,
                        P1, pb1, P2, pb2)
    return pred.reshape(_N)


# speedup vs baseline: 1.0095x; 1.0095x over previous
import dataclasses
import functools

import jax
import jax.numpy as jnp
from jax import lax
from jax.experimental import pallas as pl
from jax.experimental.pallas import tpu as pltpu
from jax.experimental.pallas import tpu_sc as plsc

_N = 10000
_E = 320000
_G = 16
_H = 64

_NC = 2
_NS = 16
_NW = _NC * _NS

_NPAD = 10112
_RPT = _NPAD // _NS
_WIN = 80
_NWIN = 125
_EHT = _E // _NW
_L = 16
_NBUF = 5


def _sc_mesh():
    return plsc.VectorSubcoreMesh(core_axis_name="c", subcore_axis_name="s")


def _sc_params(tc_tiling=True):
    cp = pltpu.CompilerParams()
    if "needs_layout_passes" in pltpu.CompilerParams.__dataclass_fields__:
        cp = dataclasses.replace(cp, needs_layout_passes=False)
    if not tc_tiling:
        cp = dataclasses.replace(cp, use_tc_tiling_on_sc=False)
    return cp


def _degree_hist(ei4):
    @functools.partial(
        pl.kernel,
        out_type=jax.ShapeDtypeStruct((_NW, _NPAD), jnp.float32),
        mesh=_sc_mesh(),
        scratch_types=[
            pltpu.VMEM((_NPAD,), jnp.float32),
            pltpu.VMEM((_NWIN, _WIN), jnp.int32),
        ],
        compiler_params=_sc_params(tc_tiling=False),
    )
    def body(ei_hbm, out_hbm, hist, dstv):
        c = lax.axis_index("c")
        s = lax.axis_index("s")
        wid = s * _NC + c
        pltpu.sync_copy(ei_hbm.at[1, wid], dstv)

        zero16 = jnp.zeros((_L,), jnp.float32)

        @pl.loop(0, _NPAD, step=_L)
        def _(i):
            hist[pl.ds(i, _L)] = zero16

        one16 = jnp.ones((_L,), jnp.float32)

        @pl.loop(0, _NWIN)
        def _(w):
            @pl.loop(0, _WIN, step=_L)
            def _(i):
                idx = dstv[w, pl.ds(i, _L)]
                plsc.addupdate_scatter(hist, [idx], one16)

        pltpu.sync_copy(hist, out_hbm.at[wid])

    return body(ei4)


def _edge_aggregate(g, ei4, zrow):
    @functools.partial(
        pl.kernel,
        out_type=jax.ShapeDtypeStruct((_NC, _NPAD, _H), jnp.float32),
        mesh=_sc_mesh(),
        scratch_types=[
            pltpu.VMEM_SHARED((_NPAD, _H), jnp.float32),
            pltpu.VMEM((_NWIN, _WIN), jnp.int32),
            pltpu.VMEM((_NWIN, _WIN), jnp.int32),
            [pltpu.VMEM((_WIN, _H), jnp.float32) for _ in range(_NBUF)],
            [pltpu.SemaphoreType.DMA for _ in range(_NBUF)],
            [pltpu.SemaphoreType.DMA for _ in range(_NBUF)],
        ],
        compiler_params=_sc_params(tc_tiling=False),
    )
    def body(g_hbm, ei_hbm, z_hbm, out_hbm,
             acc, srcv, dstv, rows, gsem, ssem):
        c = lax.axis_index("c")
        s = lax.axis_index("s")
        wid = s * _NC + c

        pltpu.sync_copy(z_hbm, acc.at[pl.ds(s * _RPT, _RPT)])
        pltpu.sync_copy(ei_hbm.at[0, wid], srcv)
        pltpu.sync_copy(ei_hbm.at[1, wid], dstv)
        plsc.subcore_barrier()

        def gather(w, b):
            pltpu.async_copy(g_hbm.at[srcv.at[w]], rows[b], gsem[b])

        def gwait(w, b):
            pltpu.make_async_copy(g_hbm.at[srcv.at[w]], rows[b], gsem[b]).wait()

        def scat(w, b):
            pltpu.async_copy(rows[b], acc.at[dstv.at[w]], ssem[b], add=True)

        def swait(w, b):
            pltpu.make_async_copy(rows[b], acc.at[dstv.at[w]], ssem[b]).wait()

        for b in range(_NBUF):
            gather(b, b)

        @pl.loop(0, _NWIN - _NBUF, step=_NBUF)
        def _(i):
            for b in range(_NBUF):
                gwait(i + b, b)
                scat(i + b, b)
            for b in range(_NBUF):
                swait(i + b, b)
                gather(i + _NBUF + b, b)

        for b in range(_NBUF):
            gwait(_NWIN - _NBUF + b, b)
            scat(_NWIN - _NBUF + b, b)
        for b in range(_NBUF):
            swait(_NWIN - _NBUF + b, b)

        plsc.subcore_barrier()
        pltpu.sync_copy(acc.at[pl.ds(s * _RPT, _RPT)],
                        out_hbm.at[c].at[pl.ds(s * _RPT, _RPT)])

    return body(g, ei4, zrow)


def _graph_norm(xv, P, cinv, w, b, ms):
    mean = jnp.dot(P.T, xv, preferred_element_type=jnp.float32) * cinv
    out = xv - jnp.dot(P, mean, preferred_element_type=jnp.float32) * ms
    var = jnp.dot(P.T, out * out, preferred_element_type=jnp.float32) * cinv
    scale = lax.rsqrt(jnp.dot(P, var, preferred_element_type=jnp.float32) + 1e-5)
    return w * out * scale + b


def _one_hot(batch2):
    P = (batch2 == lax.broadcasted_iota(jnp.int32, (1, _G), 1))
    P = P.astype(jnp.float32)
    cnt = jnp.sum(P, axis=0).reshape(_G, 1)
    return P, 1.0 / jnp.maximum(cnt, 1.0)


def _stage_a(hist_all, x, W0):
    def body(h_ref, x_ref, w_ref, dinv_ref, g0_ref):
        deg = jnp.sum(h_ref[...], axis=0) + 1.0
        dinv = lax.rsqrt(deg)
        dinv2 = dinv[:_N].reshape(_N, 1)
        dinv_ref[...] = dinv2
        h2 = jnp.dot(x_ref[...], w_ref[...], preferred_element_type=jnp.float32)
        g0_ref[...] = h2 * dinv2

    return pl.pallas_call(
        body,
        out_shape=(jax.ShapeDtypeStruct((_N, 1), jnp.float32),
                   jax.ShapeDtypeStruct((_N, _H), jnp.float32)),
    )(hist_all, x, W0)


def _stage_b(acc, g, hprev, dinv, batch2, bias, gnw, gnb, gnms, Wn):
    residual = hprev is not None

    def body(*refs):
        if residual:
            (acc_ref, g_ref, hp_ref, dinv_ref, b_ref, bias_ref,
             gnw_ref, gnb_ref, gnms_ref, wn_ref, h_ref, gn_ref) = refs
        else:
            (acc_ref, g_ref, dinv_ref, b_ref, bias_ref,
             gnw_ref, gnb_ref, gnms_ref, wn_ref, h_ref, gn_ref) = refs
        dinv2 = dinv_ref[...]
        a = acc_ref[0, :_N, :] + acc_ref[1, :_N, :] + g_ref[...]
        pre = a * dinv2 + bias_ref[...]
        P, cinv = _one_hot(b_ref[...])
        y = _graph_norm(pre, P, cinv, gnw_ref[...], gnb_ref[...], gnms_ref[...])
        h = jnp.maximum(y, 0.0)
        if residual:
            h = h + hp_ref[...]
        h_ref[...] = h
        gn_ref[...] = jnp.dot(
            h, wn_ref[...], preferred_element_type=jnp.float32) * dinv2

    args = (acc, g) + ((hprev,) if residual else ()) + (
        dinv, batch2, bias, gnw, gnb, gnms, Wn)
    return pl.pallas_call(
        body,
        out_shape=(jax.ShapeDtypeStruct((_N, _H), jnp.float32),
                   jax.ShapeDtypeStruct((_N, _H), jnp.float32)),
    )(*args)


def _stage_final(acc, g, dinv, batch2, bias, gnw, gnb, gnms, P1, pb1, P2, pb2):
    def body(acc_ref, g_ref, dinv_ref, b_ref, bias_ref,
             gnw_ref, gnb_ref, gnms_ref, p1_ref, pb1_ref, p2_ref, pb2_ref,
             out_ref):
        dinv2 = dinv_ref[...]
        a = acc_ref[0, :_N, :] + acc_ref[1, :_N, :] + g_ref[...]
        pre = a * dinv2 + bias_ref[...]
        P, cinv = _one_hot(b_ref[...])
        y = _graph_norm(pre, P, cinv, gnw_ref[...], gnb_ref[...], gnms_ref[...])
        h = jnp.maximum(y, 0.0)
        z = jnp.maximum(
            jnp.dot(h, p1_ref[...], preferred_element_type=jnp.float32)
            + pb1_ref[...], 0.0)
        t = jnp.dot(z, p2_ref[...], preferred_element_type=jnp.float32) + pb2_ref[...]
        pos = 1.0 / (1.0 + jnp.exp(-t))
        et = jnp.exp(t)
        neg = et / (1.0 + et)
        out_ref[...] = jnp.where(t >= 0, pos, neg)

    return pl.pallas_call(
        body,
        out_shape=jax.ShapeDtypeStruct((_N, 1), jnp.float32),
    )(acc, g, dinv, batch2, bias, gnw, gnb, gnms, P1, pb1, P2, pb2)


def kernel(x, edge_index, batch, W0, b0, W1, b1, W2, b2, Wl, bl,
           gn0_w, gn0_b, gn0_ms, gn1_w, gn1_b, gn1_ms, gn2_w, gn2_b, gn2_ms,
           gn3_w, gn3_b, gn3_ms, P1, pb1, P2, pb2):
    ei4 = edge_index.reshape(2, _NW, _NWIN, _WIN)
    zrow = jnp.zeros((_RPT, _H), jnp.float32)
    batch2 = batch.reshape(_N, 1)

    hist = _degree_hist(ei4)
    dinv, g = _stage_a(hist, x, W0)

    acc = _edge_aggregate(g, ei4, zrow)
    h, g = _stage_b(acc, g, None, dinv, batch2, b0, gn0_w, gn0_b, gn0_ms, W1)

    acc = _edge_aggregate(g, ei4, zrow)
    h, g = _stage_b(acc, g, h, dinv, batch2, b1, gn1_w, gn1_b, gn1_ms, W2)

    acc = _edge_aggregate(g, ei4, zrow)
    h, g = _stage_b(acc, g, h, dinv, batch2, b2, gn2_w, gn2_b, gn2_ms, Wl)

    acc = _edge_aggregate(g, ei4, zrow)
    pred = _stage_final(acc, g, dinv, batch2, bl, gn3_w, gn3_b, gn3_ms,
                        P1, pb1, P2, pb2)
    return pred.reshape(_N)
